# baseline (speedup 1.0000x reference)
import functools

import jax
import jax.numpy as jnp
from jax import lax
from jax.experimental import pallas as pl
from jax.experimental.pallas import tpu as pltpu
from jax.experimental.pallas import tpu_sc as plsc

N = 10000
E = 320000
D = 128
DOUT = 64

NC = 2
NS = 16
NW = NC * NS
CHUNK = 125
K = E // (NW * CHUNK)
STRIPE = 640
NP = NS * STRIPE
NBUF = 2
KQ = K // NBUF


def _sc_scatter_pass(table, src3, dst3, zrow, zdeg, dumr, dumd, dumi, with_deg):
    mesh = plsc.VectorSubcoreMesh(core_axis_name="c", subcore_axis_name="s")
    out_type = [jax.ShapeDtypeStruct((NC, NP, D), jnp.float32)]
    if with_deg:
        out_type.append(jax.ShapeDtypeStruct((NC, NP), jnp.float32))

    scratch = [
        pltpu.VMEM((2, NBUF, CHUNK), jnp.int32),
        pltpu.VMEM((2, NBUF, CHUNK), jnp.int32),
        pltpu.VMEM((NBUF, CHUNK, D), jnp.float32),
        pltpu.VMEM((128,), jnp.float32),
        pltpu.VMEM_SHARED((NP, D), jnp.float32),
        pltpu.VMEM_SHARED((NP,), jnp.float32),
    ] + [pltpu.SemaphoreType.DMA] * (2 + 3 * NBUF)

    def body(table_hbm, src_hbm, dst_hbm, zrow_hbm, zdeg_hbm,
             dumr_hbm, dumd_hbm, dumi_hbm, *rest):
        if with_deg:
            out_hbm, dout_hbm = rest[0], rest[1]
            scr = rest[2:]
        else:
            out_hbm = rest[0]
            scr = rest[1:]
        sring, dring, rows, ones, acc, dacc = scr[:6]
        isems = scr[6:8]
        gsems = scr[8:8 + NBUF]
        ssems = scr[8 + NBUF:8 + 2 * NBUF]
        dsems = scr[8 + 2 * NBUF:8 + 3 * NBUF]

        c = lax.axis_index("c")
        s = lax.axis_index("s")
        wid = s * NC + c
        soff = pl.multiple_of(s * STRIPE, 8)
        q0 = wid * KQ

        pltpu.sync_copy(zrow_hbm, acc.at[pl.ds(soff, STRIPE)])
        if with_deg:
            pltpu.sync_copy(zdeg_hbm, dacc.at[pl.ds(soff, STRIPE)])
            for i in range(8):
                ones[pl.ds(i * 16, 16)] = jnp.ones((16,), jnp.float32)

        def idx_load(q, p):
            pltpu.async_copy(src_hbm.at[q0 + q], sring.at[p], isems[p])
            pltpu.async_copy(dst_hbm.at[q0 + q], dring.at[p], isems[p])

        def idx_wait(p):
            pltpu.make_async_copy(dumi_hbm, sring.at[p], isems[p]).wait()
            pltpu.make_async_copy(dumi_hbm, dring.at[p], isems[p]).wait()

        def gather_start(p, b):
            pltpu.async_copy(table_hbm.at[sring.at[p, b]], rows.at[b],
                             gsems[b])

        def gather_wait(b):
            pltpu.make_async_copy(dumr_hbm, rows.at[b], gsems[b]).wait()

        def scatter_start(p, b):
            pltpu.async_copy(rows.at[b], acc.at[dring.at[p, b]], ssems[b],
                             add=True)
            if with_deg:
                pltpu.async_copy(ones.at[pl.ds(0, CHUNK)],
                                 dacc.at[dring.at[p, b]], dsems[b], add=True)

        def scatter_wait(b):
            pltpu.make_async_copy(dumr_hbm, rows.at[b], ssems[b]).wait()
            if with_deg:
                pltpu.make_async_copy(
                    dumd_hbm, ones.at[pl.ds(0, CHUNK)], dsems[b]).wait()

        def do_quad(q, p, load_next, start_next):
            for b in range(NBUF):
                gather_wait(b)
                scatter_start(p, b)
            for b in range(NBUF):
                scatter_wait(b)
                if start_next:
                    if b == 0:
                        idx_wait(1 - p)
                    gather_start(1 - p, b)
            if load_next:
                idx_load(q + 2, p)

        idx_load(0, 0)
        idx_load(1, 1)
        plsc.subcore_barrier()
        idx_wait(0)
        for b in range(NBUF):
            gather_start(0, b)

        def pair_body(i, _):
            do_quad(2 * i, 0, True, True)
            do_quad(2 * i + 1, 1, True, True)
            return 0
        lax.fori_loop(0, KQ // 2 - 1, pair_body, 0)

        do_quad(KQ - 2, 0, False, True)
        do_quad(KQ - 1, 1, False, False)

        plsc.subcore_barrier()

        sl = pl.ds(soff, STRIPE)
        pltpu.sync_copy(acc.at[sl], out_hbm.at[c, sl])
        if with_deg:
            pltpu.sync_copy(
                dacc.at[sl],
                dout_hbm.at[c, pl.ds(pl.multiple_of(s * STRIPE, 128), STRIPE)])

    run = pl.kernel(body, out_type=out_type, mesh=mesh, scratch_types=scratch)
    return run(table, src3, dst3, zrow, zdeg, dumr, dumd, dumi)


def _tc_dense_a(x, parts, dparts, W1l, b1, W1r, Wcat):
    R = 1000
    grid = (N // R,)

    def body(x_ref, p_ref, d_ref, wl_ref, b1_ref, wr_ref, wc_ref,
             hw_ref, hs_ref):
        deg = d_ref[0] + d_ref[1]
        agg = (p_ref[0] + p_ref[1]) / jnp.maximum(deg, 1.0)
        h = agg @ wl_ref[...] + b1_ref[...] + x_ref[...] @ wr_ref[...]
        h = jnp.maximum(h, 0.0)
        hw = h @ wc_ref[...]
        hw_ref[...] = hw
        hs_ref[...] = hw * lax.rsqrt(deg + 1.0)

    return pl.pallas_call(
        body,
        grid=grid,
        in_specs=[
            pl.BlockSpec((R, D), lambda i: (i, 0)),
            pl.BlockSpec((NC, R, D), lambda i: (0, i, 0)),
            pl.BlockSpec((NC, R, 1), lambda i: (0, i, 0)),
            pl.BlockSpec((D, D), lambda i: (0, 0)),
            pl.BlockSpec((1, D), lambda i: (0, 0)),
            pl.BlockSpec((D, D), lambda i: (0, 0)),
            pl.BlockSpec((D, D), lambda i: (0, 0)),
        ],
        out_specs=[
            pl.BlockSpec((R, D), lambda i: (i, 0)),
            pl.BlockSpec((R, D), lambda i: (i, 0)),
        ],
        out_shape=[
            jax.ShapeDtypeStruct((N, D), jnp.float32),
            jax.ShapeDtypeStruct((N, D), jnp.float32),
        ],
    )(x, parts, dparts, W1l, b1, W1r, Wcat)


def _tc_dense_b(parts2, hw, dparts, bmu, bls):
    R = 1000
    grid = (N // R,)

    def body(p_ref, hw_ref, d_ref, bmu_ref, bls_ref, mu_ref, ls_ref):
        deg = d_ref[0] + d_ref[1] + 1.0
        dinv = lax.rsqrt(deg)
        o = (p_ref[0] + p_ref[1]) * dinv + hw_ref[...] * (1.0 / deg)
        mu_ref[...] = o[:, :DOUT] + bmu_ref[...]
        ls_ref[...] = o[:, DOUT:] + bls_ref[...]

    return pl.pallas_call(
        body,
        grid=grid,
        in_specs=[
            pl.BlockSpec((NC, R, D), lambda i: (0, i, 0)),
            pl.BlockSpec((R, D), lambda i: (i, 0)),
            pl.BlockSpec((NC, R, 1), lambda i: (0, i, 0)),
            pl.BlockSpec((1, DOUT), lambda i: (0, 0)),
            pl.BlockSpec((1, DOUT), lambda i: (0, 0)),
        ],
        out_specs=[
            pl.BlockSpec((R, DOUT), lambda i: (i, 0)),
            pl.BlockSpec((R, DOUT), lambda i: (i, 0)),
        ],
        out_shape=[
            jax.ShapeDtypeStruct((N, DOUT), jnp.float32),
            jax.ShapeDtypeStruct((N, DOUT), jnp.float32),
        ],
    )(parts2, hw, dparts, bmu, bls)


def kernel(x, edge_index, W1l, b1, W1r, Wmu, bmu, Wls, bls):
    src3 = edge_index[0].reshape(E // (NBUF * CHUNK), NBUF, CHUNK)
    dst3 = edge_index[1].reshape(E // (NBUF * CHUNK), NBUF, CHUNK)
    zrow = jnp.zeros((STRIPE, D), jnp.float32)
    zdeg = jnp.zeros((STRIPE,), jnp.float32)
    dumr = jnp.zeros((CHUNK, D), jnp.float32)
    dumd = jnp.zeros((CHUNK,), jnp.float32)
    dumi = jnp.zeros((NBUF, CHUNK), jnp.int32)
    Wcat = jnp.concatenate([Wmu, Wls], axis=1)

    parts, dparts = _sc_scatter_pass(x, src3, dst3, zrow, zdeg, dumr, dumd, dumi, with_deg=True)
    dparts = dparts.reshape(NC, NP, 1)
    hw, hs = _tc_dense_a(x, parts, dparts, W1l, b1.reshape(1, D), W1r, Wcat)
    (parts2,) = _sc_scatter_pass(hs, src3, dst3, zrow, zdeg, dumr, dumd, dumi, with_deg=False)
    mu, logstd = _tc_dense_b(parts2, hw, dparts,
                             bmu.reshape(1, DOUT), bls.reshape(1, DOUT))
    return (mu, logstd)


# speedup vs baseline: 1.2047x; 1.2047x over previous
import functools

import jax
import jax.numpy as jnp
from jax import lax
from jax.experimental import pallas as pl
from jax.experimental.pallas import tpu as pltpu
from jax.experimental.pallas import tpu_sc as plsc

N = 10000
E = 320000
D = 128
DOUT = 64

NC = 2
NS = 16
NW = NC * NS
CHUNK = 128
EP = 2560 * CHUNK
K = EP // (NW * CHUNK)
SQ = 8
KQ = K // SQ
STRIPE = 640
NP = NS * STRIPE
NBUF = 2


def _sc_scatter_pass(table, epad3, zrow, zdeg, dumr, dumd, dumi, with_deg):
    mesh = plsc.VectorSubcoreMesh(core_axis_name="c", subcore_axis_name="s")
    out_type = [jax.ShapeDtypeStruct((NC, NP, D), jnp.float32)]
    if with_deg:
        out_type.append(jax.ShapeDtypeStruct((NC, NP), jnp.float32))

    scratch = [
        pltpu.VMEM((2, SQ, CHUNK), jnp.int32),
        pltpu.VMEM((2, SQ, CHUNK), jnp.int32),
        pltpu.VMEM((NBUF, CHUNK, D), jnp.float32),
        pltpu.VMEM((CHUNK,), jnp.float32),
        pltpu.VMEM_SHARED((NP, D), jnp.float32),
        pltpu.VMEM_SHARED((NP,), jnp.float32),
    ] + [pltpu.SemaphoreType.DMA] * (2 + 3 * NBUF)

    def body(table_hbm, eidx_hbm, zrow_hbm, zdeg_hbm,
             dumr_hbm, dumd_hbm, dumi_hbm, *rest):
        if with_deg:
            out_hbm, dout_hbm = rest[0], rest[1]
            scr = rest[2:]
        else:
            out_hbm = rest[0]
            scr = rest[1:]
        sring, dring, rows, ones, acc, dacc = scr[:6]
        isems = scr[6:8]
        gsems = scr[8:8 + NBUF]
        ssems = scr[8 + NBUF:8 + 2 * NBUF]
        dsems = scr[8 + 2 * NBUF:8 + 3 * NBUF]

        c = lax.axis_index("c")
        s = lax.axis_index("s")
        wid = s * NC + c
        soff = pl.multiple_of(s * STRIPE, 8)
        row0 = wid * K

        pltpu.sync_copy(zrow_hbm, acc.at[pl.ds(soff, STRIPE)])
        if with_deg:
            pltpu.sync_copy(zdeg_hbm, dacc.at[pl.ds(soff, STRIPE)])
            for i in range(8):
                ones[pl.ds(i * 16, 16)] = jnp.ones((16,), jnp.float32)

        def idx_load(g, p):
            ro = pl.multiple_of(row0 + g * SQ, 8)
            pltpu.async_copy(eidx_hbm.at[0, pl.ds(ro, SQ)], sring.at[p],
                             isems[p])
            pltpu.async_copy(eidx_hbm.at[1, pl.ds(ro, SQ)], dring.at[p],
                             isems[p])

        def idx_wait(p):
            pltpu.make_async_copy(dumi_hbm, sring.at[p], isems[p]).wait()
            pltpu.make_async_copy(dumi_hbm, dring.at[p], isems[p]).wait()

        def gather_start(p, r, b):
            pltpu.async_copy(table_hbm.at[sring.at[p, r]], rows.at[b],
                             gsems[b])

        def gather_wait(b):
            pltpu.make_async_copy(dumr_hbm, rows.at[b], gsems[b]).wait()

        def scatter_start(p, r, b):
            pltpu.async_copy(rows.at[b], acc.at[dring.at[p, r]], ssems[b],
                             add=True)
            if with_deg:
                pltpu.async_copy(ones, dacc.at[dring.at[p, r]], dsems[b],
                                 add=True)

        def scatter_wait(b):
            pltpu.make_async_copy(dumr_hbm, rows.at[b], ssems[b]).wait()
            if with_deg:
                pltpu.make_async_copy(dumd_hbm, ones, dsems[b]).wait()

        def step(j, u, issue_next, load_next):
            b = u % NBUF
            g = u // SQ
            p = g % 2
            r = u % SQ
            gather_wait(b)
            scatter_start(p, r, b)
            scatter_wait(b)
            u2 = u + NBUF
            if issue_next:
                p2 = (u2 // SQ) % 2
                if r == SQ - NBUF:
                    idx_wait(p2)
                gather_start(p2, u2 % SQ, b)
            if r == SQ - 1 and load_next:
                idx_load(j + u // SQ + 2, p)

        idx_load(0, 0)
        idx_load(1, 1)
        plsc.subcore_barrier()
        idx_wait(0)
        for b in range(NBUF):
            gather_start(0, b, b)

        UNROLL = 2 * SQ
        def pair_body(i, _):
            g2 = 2 * i
            for u in range(UNROLL):
                step(g2, u, True, True)
            return 0
        lax.fori_loop(0, KQ // 2 - 1, pair_body, 0)

        for u in range(UNROLL):
            j = KQ - 2
            step(j, u, u < UNROLL - NBUF, False)

        plsc.subcore_barrier()

        sl = pl.ds(soff, STRIPE)
        pltpu.sync_copy(acc.at[sl], out_hbm.at[c, sl])
        if with_deg:
            pltpu.sync_copy(
                dacc.at[sl],
                dout_hbm.at[c, pl.ds(pl.multiple_of(s * STRIPE, 128), STRIPE)])

    run = pl.kernel(body, out_type=out_type, mesh=mesh, scratch_types=scratch)
    return run(table, epad3, zrow, zdeg, dumr, dumd, dumi)


def _tc_dense_a(x, parts, dparts, W1l, b1, W1r, Wcat):
    R = 1000
    grid = (N // R,)

    def body(x_ref, p_ref, d_ref, wl_ref, b1_ref, wr_ref, wc_ref,
             hw_ref, hs_ref):
        deg = d_ref[0] + d_ref[1]
        agg = (p_ref[0] + p_ref[1]) / jnp.maximum(deg, 1.0)
        h = agg @ wl_ref[...] + b1_ref[...] + x_ref[...] @ wr_ref[...]
        h = jnp.maximum(h, 0.0)
        hw = h @ wc_ref[...]
        hw_ref[...] = hw
        hs_ref[...] = hw * lax.rsqrt(deg + 1.0)

    return pl.pallas_call(
        body,
        grid=grid,
        in_specs=[
            pl.BlockSpec((R, D), lambda i: (i, 0)),
            pl.BlockSpec((NC, R, D), lambda i: (0, i, 0)),
            pl.BlockSpec((NC, R, 1), lambda i: (0, i, 0)),
            pl.BlockSpec((D, D), lambda i: (0, 0)),
            pl.BlockSpec((1, D), lambda i: (0, 0)),
            pl.BlockSpec((D, D), lambda i: (0, 0)),
            pl.BlockSpec((D, D), lambda i: (0, 0)),
        ],
        out_specs=[
            pl.BlockSpec((R, D), lambda i: (i, 0)),
            pl.BlockSpec((R, D), lambda i: (i, 0)),
        ],
        out_shape=[
            jax.ShapeDtypeStruct((N, D), jnp.float32),
            jax.ShapeDtypeStruct((N, D), jnp.float32),
        ],
    )(x, parts, dparts, W1l, b1, W1r, Wcat)


def _tc_dense_b(parts2, hw, dparts, bmu, bls):
    R = 1000
    grid = (N // R,)

    def body(p_ref, hw_ref, d_ref, bmu_ref, bls_ref, mu_ref, ls_ref):
        deg = d_ref[0] + d_ref[1] + 1.0
        dinv = lax.rsqrt(deg)
        o = (p_ref[0] + p_ref[1]) * dinv + hw_ref[...] * (1.0 / deg)
        mu_ref[...] = o[:, :DOUT] + bmu_ref[...]
        ls_ref[...] = o[:, DOUT:] + bls_ref[...]

    return pl.pallas_call(
        body,
        grid=grid,
        in_specs=[
            pl.BlockSpec((NC, R, D), lambda i: (0, i, 0)),
            pl.BlockSpec((R, D), lambda i: (i, 0)),
            pl.BlockSpec((NC, R, 1), lambda i: (0, i, 0)),
            pl.BlockSpec((1, DOUT), lambda i: (0, 0)),
            pl.BlockSpec((1, DOUT), lambda i: (0, 0)),
        ],
        out_specs=[
            pl.BlockSpec((R, DOUT), lambda i: (i, 0)),
            pl.BlockSpec((R, DOUT), lambda i: (i, 0)),
        ],
        out_shape=[
            jax.ShapeDtypeStruct((N, DOUT), jnp.float32),
            jax.ShapeDtypeStruct((N, DOUT), jnp.float32),
        ],
    )(parts2, hw, dparts, bmu, bls)


def kernel(x, edge_index, W1l, b1, W1r, Wmu, bmu, Wls, bls):
    npad = EP - E
    pad_src = (jnp.arange(npad, dtype=jnp.int32) * 37) % N
    pad_dst = N + (jnp.arange(npad, dtype=jnp.int32) % (NP - N))
    epad3 = jnp.concatenate(
        [edge_index, jnp.stack([pad_src, pad_dst])], axis=1
    ).reshape(2, EP // CHUNK, CHUNK)
    zrow = jnp.zeros((STRIPE, D), jnp.float32)
    zdeg = jnp.zeros((STRIPE,), jnp.float32)
    dumr = jnp.zeros((CHUNK, D), jnp.float32)
    dumd = jnp.zeros((CHUNK,), jnp.float32)
    dumi = jnp.zeros((SQ, CHUNK), jnp.int32)
    Wcat = jnp.concatenate([Wmu, Wls], axis=1)

    parts, dparts = _sc_scatter_pass(x, epad3, zrow, zdeg, dumr, dumd, dumi,
                                     with_deg=True)
    dparts = dparts.reshape(NC, NP, 1)
    hw, hs = _tc_dense_a(x, parts, dparts, W1l, b1.reshape(1, D), W1r, Wcat)
    (parts2,) = _sc_scatter_pass(hs, epad3, zrow, zdeg, dumr, dumd, dumi,
                                 with_deg=False)
    mu, logstd = _tc_dense_b(parts2, hw, dparts,
                             bmu.reshape(1, DOUT), bls.reshape(1, DOUT))
    return (mu, logstd)


# speedup vs baseline: 1.2099x; 1.0043x over previous
import functools

import numpy as np

import jax
import jax.numpy as jnp
from jax import lax
from jax.experimental import pallas as pl
from jax.experimental.pallas import tpu as pltpu
from jax.experimental.pallas import tpu_sc as plsc

N = 10000
E = 320000
D = 128
DOUT = 64

NC = 2
NS = 16
NW = NC * NS
CHUNK = 128
EP = 2560 * CHUNK
K = EP // (NW * CHUNK)
SQ = 8
KQ = K // SQ
STRIPE = 640
NP = NS * STRIPE
NBUF = 2


def _sc_scatter_pass(table, epad3, zrow, zdeg, dumr, dumd, dumi, with_deg):
    mesh = plsc.VectorSubcoreMesh(core_axis_name="c", subcore_axis_name="s")
    out_type = [jax.ShapeDtypeStruct((NC, NP, D), jnp.float32)]
    if with_deg:
        out_type.append(jax.ShapeDtypeStruct((NC, NP), jnp.float32))

    scratch = [
        pltpu.VMEM((2, SQ, CHUNK), jnp.int32),
        pltpu.VMEM((2, SQ, CHUNK), jnp.int32),
        pltpu.VMEM((NBUF, CHUNK, D), jnp.float32),
        pltpu.VMEM((CHUNK,), jnp.float32),
        pltpu.VMEM_SHARED((NP, D), jnp.float32),
        pltpu.VMEM_SHARED((NP,), jnp.float32),
    ] + [pltpu.SemaphoreType.DMA] * (3 + 2 * NBUF)

    def body(table_hbm, eidx_hbm, zrow_hbm, zdeg_hbm,
             dumr_hbm, dumd_hbm, dumi_hbm, *rest):
        if with_deg:
            out_hbm, dout_hbm = rest[0], rest[1]
            scr = rest[2:]
        else:
            out_hbm = rest[0]
            scr = rest[1:]
        sring, dring, rows, ones, acc, dacc = scr[:6]
        isems = scr[6:8]
        gsems = scr[8:8 + NBUF]
        ssems = scr[8 + NBUF:8 + 2 * NBUF]
        dsem = scr[8 + 2 * NBUF]

        c = lax.axis_index("c")
        s = lax.axis_index("s")
        wid = s * NC + c
        soff = pl.multiple_of(s * STRIPE, 8)
        row0 = wid * K

        pltpu.sync_copy(zrow_hbm, acc.at[pl.ds(soff, STRIPE)])
        if with_deg:
            pltpu.sync_copy(zdeg_hbm, dacc.at[pl.ds(soff, STRIPE)])
            for i in range(8):
                ones[pl.ds(i * 16, 16)] = jnp.ones((16,), jnp.float32)

        def idx_load(g, p):
            ro = pl.multiple_of(row0 + g * SQ, 8)
            pltpu.async_copy(eidx_hbm.at[0, pl.ds(ro, SQ)], sring.at[p],
                             isems[p])
            pltpu.async_copy(eidx_hbm.at[1, pl.ds(ro, SQ)], dring.at[p],
                             isems[p])

        def idx_wait(p):
            pltpu.make_async_copy(dumi_hbm, sring.at[p], isems[p]).wait()
            pltpu.make_async_copy(dumi_hbm, dring.at[p], isems[p]).wait()

        def gather_start(p, r, b):
            pltpu.async_copy(table_hbm.at[sring.at[p, r]], rows.at[b],
                             gsems[b])

        def gather_wait(b):
            pltpu.make_async_copy(dumr_hbm, rows.at[b], gsems[b]).wait()

        def scatter_start(p, r, b):
            pltpu.async_copy(rows.at[b], acc.at[dring.at[p, r]], ssems[b],
                             add=True)
            if with_deg:
                pltpu.async_copy(ones, dacc.at[dring.at[p, r]], dsem,
                                 add=True)

        def scatter_wait(b):
            pltpu.make_async_copy(dumr_hbm, rows.at[b], ssems[b]).wait()

        def step(j, u, issue_next, load_next):
            b = u % NBUF
            g = u // SQ
            p = g % 2
            r = u % SQ
            gather_wait(b)
            scatter_start(p, r, b)
            scatter_wait(b)
            u2 = u + NBUF
            if issue_next:
                p2 = (u2 // SQ) % 2
                if r == SQ - NBUF:
                    idx_wait(p2)
                gather_start(p2, u2 % SQ, b)
            if r == SQ - 1 and load_next:
                idx_load(j + u // SQ + 2, p)

        idx_load(0, 0)
        idx_load(1, 1)
        plsc.subcore_barrier()
        idx_wait(0)
        for b in range(NBUF):
            gather_start(0, b, b)

        UNROLL = 2 * SQ
        def pair_body(i, _):
            g2 = 2 * i
            for u in range(UNROLL):
                step(g2, u, True, True)
            return 0
        lax.fori_loop(0, KQ // 2 - 1, pair_body, 0)

        for u in range(UNROLL):
            j = KQ - 2
            step(j, u, u < UNROLL - NBUF, False)

        if with_deg:
            def deg_drain(i, _):
                pltpu.make_async_copy(dumd_hbm, ones, dsem).wait()
                return 0
            lax.fori_loop(0, K, deg_drain, 0)

        plsc.subcore_barrier()

        sl = pl.ds(soff, STRIPE)
        pltpu.sync_copy(acc.at[sl], out_hbm.at[c, sl])
        if with_deg:
            pltpu.sync_copy(
                dacc.at[sl],
                dout_hbm.at[c, pl.ds(pl.multiple_of(s * STRIPE, 128), STRIPE)])

    run = pl.kernel(body, out_type=out_type, mesh=mesh, scratch_types=scratch)
    return run(table, epad3, zrow, zdeg, dumr, dumd, dumi)


def _tc_dense_a(x, parts, dparts, W1l, b1, W1r, Wcat):
    R = 1000
    grid = (N // R,)

    def body(x_ref, p_ref, d_ref, wl_ref, b1_ref, wr_ref, wc_ref,
             hw_ref, hs_ref):
        deg = d_ref[0] + d_ref[1]
        agg = (p_ref[0] + p_ref[1]) / jnp.maximum(deg, 1.0)
        h = agg @ wl_ref[...] + b1_ref[...] + x_ref[...] @ wr_ref[...]
        h = jnp.maximum(h, 0.0)
        hw = h @ wc_ref[...]
        hw_ref[...] = hw
        hs_ref[...] = hw * lax.rsqrt(deg + 1.0)

    return pl.pallas_call(
        body,
        grid=grid,
        in_specs=[
            pl.BlockSpec((R, D), lambda i: (i, 0)),
            pl.BlockSpec((NC, R, D), lambda i: (0, i, 0)),
            pl.BlockSpec((NC, R, 1), lambda i: (0, i, 0)),
            pl.BlockSpec((D, D), lambda i: (0, 0)),
            pl.BlockSpec((1, D), lambda i: (0, 0)),
            pl.BlockSpec((D, D), lambda i: (0, 0)),
            pl.BlockSpec((D, D), lambda i: (0, 0)),
        ],
        out_specs=[
            pl.BlockSpec((R, D), lambda i: (i, 0)),
            pl.BlockSpec((R, D), lambda i: (i, 0)),
        ],
        out_shape=[
            jax.ShapeDtypeStruct((N, D), jnp.float32),
            jax.ShapeDtypeStruct((N, D), jnp.float32),
        ],
    )(x, parts, dparts, W1l, b1, W1r, Wcat)


def _tc_dense_b(parts2, hw, dparts, bmu, bls):
    R = 1000
    grid = (N // R,)

    def body(p_ref, hw_ref, d_ref, bmu_ref, bls_ref, mu_ref, ls_ref):
        deg = d_ref[0] + d_ref[1] + 1.0
        dinv = lax.rsqrt(deg)
        o = (p_ref[0] + p_ref[1]) * dinv + hw_ref[...] * (1.0 / deg)
        mu_ref[...] = o[:, :DOUT] + bmu_ref[...]
        ls_ref[...] = o[:, DOUT:] + bls_ref[...]

    return pl.pallas_call(
        body,
        grid=grid,
        in_specs=[
            pl.BlockSpec((NC, R, D), lambda i: (0, i, 0)),
            pl.BlockSpec((R, D), lambda i: (i, 0)),
            pl.BlockSpec((NC, R, 1), lambda i: (0, i, 0)),
            pl.BlockSpec((1, DOUT), lambda i: (0, 0)),
            pl.BlockSpec((1, DOUT), lambda i: (0, 0)),
        ],
        out_specs=[
            pl.BlockSpec((R, DOUT), lambda i: (i, 0)),
            pl.BlockSpec((R, DOUT), lambda i: (i, 0)),
        ],
        out_shape=[
            jax.ShapeDtypeStruct((N, DOUT), jnp.float32),
            jax.ShapeDtypeStruct((N, DOUT), jnp.float32),
        ],
    )(parts2, hw, dparts, bmu, bls)


_PAD_N = EP - E
_PAD_EDGES = jnp.asarray(np.stack([
    (np.arange(_PAD_N, dtype=np.int32) * 37) % N,
    N + (np.arange(_PAD_N, dtype=np.int32) % (NP - N)),
]))


def kernel(x, edge_index, W1l, b1, W1r, Wmu, bmu, Wls, bls):
    epad3 = jnp.concatenate(
        [edge_index, _PAD_EDGES], axis=1).reshape(2, EP // CHUNK, CHUNK)
    zrow = jnp.zeros((STRIPE, D), jnp.float32)
    zdeg = jnp.zeros((STRIPE,), jnp.float32)
    dumr = jnp.zeros((CHUNK, D), jnp.float32)
    dumd = jnp.zeros((CHUNK,), jnp.float32)
    dumi = jnp.zeros((SQ, CHUNK), jnp.int32)
    Wcat = jnp.concatenate([Wmu, Wls], axis=1)

    parts, dparts = _sc_scatter_pass(x, epad3, zrow, zdeg, dumr, dumd, dumi,
                                     with_deg=True)
    dparts = dparts.reshape(NC, NP, 1)
    hw, hs = _tc_dense_a(x, parts, dparts, W1l, b1.reshape(1, D), W1r, Wcat)
    (parts2,) = _sc_scatter_pass(hs, epad3, zrow, zdeg, dumr, dumd, dumi,
                                 with_deg=False)
    mu, logstd = _tc_dense_b(parts2, hw, dparts,
                             bmu.reshape(1, DOUT), bls.reshape(1, DOUT))
    return (mu, logstd)
